# baseline (speedup 1.0000x reference)
import math

import jax
import jax.numpy as jnp
from jax.experimental import pallas as pl


_SQRT_2_OVER_PI = math.sqrt(2.0 / math.pi)


def _gelu_block(x_ref, o_ref):
    x = x_ref[...]
    z = _SQRT_2_OVER_PI * (x + 0.044715 * (x * x * x))
    z2 = z * z
    t = z * (945.0 + z2 * (105.0 + z2)) / (945.0 + z2 * (420.0 + 15.0 * z2))
    t = jnp.clip(t, -1.0, 1.0)
    o_ref[...] = 0.5 * x * (1.0 + t)


def kernel(x, log_k_local, log_k_global):
    B, T, D = x.shape
    rows = B * T
    x2 = x.reshape(rows, D)
    block_rows = 2048
    grid = (rows // block_rows,)
    y = pl.pallas_call(
        _gelu_block,
        grid=grid,
        in_specs=[pl.BlockSpec((block_rows, D), lambda i: (i, 0))],
        out_specs=pl.BlockSpec((block_rows, D), lambda i: (i, 0)),
        out_shape=jax.ShapeDtypeStruct((rows, D), x.dtype),
    )(x2)
    return y.reshape(B, T, D)


# speedup vs baseline: 1.1805x; 1.1805x over previous
import math

import jax
import jax.numpy as jnp
from jax.experimental import pallas as pl


_SQRT_2_OVER_PI = math.sqrt(2.0 / math.pi)
_C2 = 0.044715 * math.sqrt(2.0 / math.pi)


def _gelu_block(x_ref, o_ref):
    x = x_ref[...]
    z = x * (_SQRT_2_OVER_PI + _C2 * (x * x))
    t = jnp.tanh(z)
    h = 0.5 * x
    o_ref[...] = h + h * t


def kernel(x, log_k_local, log_k_global):
    B, T, D = x.shape
    rows = B * T
    x2 = x.reshape(rows, D)
    block_rows = 2048
    grid = (rows // block_rows,)
    y = pl.pallas_call(
        _gelu_block,
        grid=grid,
        in_specs=[pl.BlockSpec((block_rows, D), lambda i: (i, 0))],
        out_specs=pl.BlockSpec((block_rows, D), lambda i: (i, 0)),
        out_shape=jax.ShapeDtypeStruct((rows, D), x.dtype),
    )(x2)
    return y.reshape(B, T, D)


# speedup vs baseline: 1.2175x; 1.0313x over previous
import functools
import math

import jax
import jax.numpy as jnp
from jax.experimental import pallas as pl
from jax.experimental.pallas import tpu as pltpu


_SQRT_2_OVER_PI = math.sqrt(2.0 / math.pi)
_C2 = 0.044715 * math.sqrt(2.0 / math.pi)

_NBUF = 4
_CHUNK_ROWS = 512


def _gelu(x):
    z = x * (_SQRT_2_OVER_PI + _C2 * (x * x))
    t = jnp.tanh(z)
    h = 0.5 * x
    return h + h * t


def _pipelined_body(n_chunks, x_hbm, o_hbm, in_buf, out_buf, in_sem, out_sem):
    cr = _CHUNK_ROWS

    def start_in(i, b):
        pltpu.make_async_copy(
            x_hbm.at[pl.ds(i * cr, cr)], in_buf.at[b], in_sem.at[b]
        ).start()

    for k in range(_NBUF):
        start_in(k, k)

    def loop_body(i, carry):
        b = jax.lax.rem(i, _NBUF)
        pltpu.make_async_copy(
            x_hbm.at[pl.ds(i * cr, cr)], in_buf.at[b], in_sem.at[b]
        ).wait()

        @pl.when(i >= _NBUF)
        def _():
            pltpu.make_async_copy(
                out_buf.at[b], o_hbm.at[pl.ds((i - _NBUF) * cr, cr)], out_sem.at[b]
            ).wait()

        out_buf[b] = _gelu(in_buf[b])
        pltpu.make_async_copy(
            out_buf.at[b], o_hbm.at[pl.ds(i * cr, cr)], out_sem.at[b]
        ).start()

        @pl.when(i + _NBUF < n_chunks)
        def _():
            start_in(i + _NBUF, b)

        return carry

    jax.lax.fori_loop(0, n_chunks, loop_body, 0)

    for k in range(_NBUF):
        i = n_chunks - _NBUF + k
        pltpu.make_async_copy(
            out_buf.at[i % _NBUF], o_hbm.at[pl.ds(i * cr, cr)], out_sem.at[i % _NBUF]
        ).wait()


def kernel(x, log_k_local, log_k_global):
    B, T, D = x.shape
    rows = B * T
    n_chunks = rows // _CHUNK_ROWS
    x2 = x.reshape(rows, D)
    y = pl.pallas_call(
        functools.partial(_pipelined_body, n_chunks),
        in_specs=[pl.BlockSpec(memory_space=pltpu.MemorySpace.HBM)],
        out_specs=pl.BlockSpec(memory_space=pltpu.MemorySpace.HBM),
        out_shape=jax.ShapeDtypeStruct((rows, D), x.dtype),
        scratch_shapes=[
            pltpu.VMEM((_NBUF, _CHUNK_ROWS, D), x.dtype),
            pltpu.VMEM((_NBUF, _CHUNK_ROWS, D), x.dtype),
            pltpu.SemaphoreType.DMA((_NBUF,)),
            pltpu.SemaphoreType.DMA((_NBUF,)),
        ],
    )(x2)
    return y.reshape(B, T, D)


# speedup vs baseline: 1.2201x; 1.0022x over previous
import functools
import math

import jax
import jax.numpy as jnp
from jax.experimental import pallas as pl
from jax.experimental.pallas import tpu as pltpu


_SQRT_2_OVER_PI = math.sqrt(2.0 / math.pi)
_C2 = 0.044715 * math.sqrt(2.0 / math.pi)

_NBUF = 8
_CHUNK_ROWS = 256


def _gelu(x):
    z = x * (_SQRT_2_OVER_PI + _C2 * (x * x))
    t = jnp.tanh(z)
    h = 0.5 * x
    return h + h * t


def _pipelined_body(n_chunks, x_hbm, o_hbm, in_buf, out_buf, in_sem, out_sem):
    cr = _CHUNK_ROWS

    def start_in(i, b):
        pltpu.make_async_copy(
            x_hbm.at[pl.ds(i * cr, cr)], in_buf.at[b], in_sem.at[b]
        ).start()

    for k in range(_NBUF):
        start_in(k, k)

    def loop_body(i, carry):
        b = jax.lax.rem(i, _NBUF)
        pltpu.make_async_copy(
            x_hbm.at[pl.ds(i * cr, cr)], in_buf.at[b], in_sem.at[b]
        ).wait()

        @pl.when(i >= _NBUF)
        def _():
            pltpu.make_async_copy(
                out_buf.at[b], o_hbm.at[pl.ds((i - _NBUF) * cr, cr)], out_sem.at[b]
            ).wait()

        out_buf[b] = _gelu(in_buf[b])
        pltpu.make_async_copy(
            out_buf.at[b], o_hbm.at[pl.ds(i * cr, cr)], out_sem.at[b]
        ).start()

        @pl.when(i + _NBUF < n_chunks)
        def _():
            start_in(i + _NBUF, b)

        return carry

    jax.lax.fori_loop(0, n_chunks, loop_body, 0)

    for k in range(_NBUF):
        i = n_chunks - _NBUF + k
        pltpu.make_async_copy(
            out_buf.at[i % _NBUF], o_hbm.at[pl.ds(i * cr, cr)], out_sem.at[i % _NBUF]
        ).wait()


def kernel(x, log_k_local, log_k_global):
    B, T, D = x.shape
    rows = B * T
    n_chunks = rows // _CHUNK_ROWS
    x2 = x.reshape(rows, D)
    y = pl.pallas_call(
        functools.partial(_pipelined_body, n_chunks),
        in_specs=[pl.BlockSpec(memory_space=pltpu.MemorySpace.HBM)],
        out_specs=pl.BlockSpec(memory_space=pltpu.MemorySpace.HBM),
        out_shape=jax.ShapeDtypeStruct((rows, D), x.dtype),
        scratch_shapes=[
            pltpu.VMEM((_NBUF, _CHUNK_ROWS, D), x.dtype),
            pltpu.VMEM((_NBUF, _CHUNK_ROWS, D), x.dtype),
            pltpu.SemaphoreType.DMA((_NBUF,)),
            pltpu.SemaphoreType.DMA((_NBUF,)),
        ],
    )(x2)
    return y.reshape(B, T, D)
